# baseline (speedup 1.0000x reference)
import functools

import jax
import jax.numpy as jnp
import numpy as np
from jax import lax
from jax.experimental import pallas as pl
from jax.experimental.pallas import tpu as pltpu
from jax.experimental.pallas import tpu_sc as plsc

_K = 16
_NS = float(1.0 / np.sqrt(1.0 + 1e-5))
_F32 = jnp.float32


def _leaky(v):
    return jnp.where(v >= 0, v, v * 0.2)


def _fold(p):
    W = p['W']
    b = p.get('b')
    if 'gamma' in p:
        g = p['gamma'] * _NS
        W = W * g[None, :]
        b = (b * g + p['beta']) if b is not None else p['beta']
    return W, (None if b is None else b.reshape(1, -1))


def _dot(a, b):
    return jnp.dot(a, b, preferred_element_type=_F32)


def _pre_body(n, tq, has_fc0, pos_r, posT_r, x_r, *rest):
    nbr_r, sc_r, tab_r = rest[-3], rest[-2], rest[-1]
    it = iter(rest[:-3])
    nxt = lambda: next(it)[...]

    x = x_r[...]
    if has_fc0:
        x = _dot(x, nxt()) + nxt()
    sc_r[...] = _dot(x, nxt()) + nxt()
    h = _leaky(_dot(x, nxt()) + nxt())
    ch = h.shape[1]
    tab_r[:, 0:ch] = h
    tab_r[:, ch:ch + 3] = pos_r[:, 0:3]
    tab_r[:, ch + 3:] = jnp.zeros((n, 128 - ch - 3), _F32)

    iota = lax.broadcasted_iota(jnp.int32, (tq, n), 1)

    def tile(t, carry):
        base = t * tq
        q = pos_r[pl.ds(base, tq), :]
        d = None
        for c in range(3):
            df = q[:, c:c + 1] - posT_r[c:c + 1, :]
            d = df * df if d is None else d + df * df
        for j in range(_K):
            dmin = jnp.min(d, axis=1, keepdims=True)
            eqm = d == dmin
            idxc = jnp.min(jnp.where(eqm, iota, n), axis=1, keepdims=True)
            nbr_r[pl.ds(base, tq), pl.ds(j, 1)] = idxc
            d = jnp.where(eqm, jnp.float32(jnp.inf), d)
        return carry

    lax.fori_loop(0, n // tq, tile, 0)


def _pre(pos_p, posT, x, warrs, n, dout, has_fc0):
    tq = min(n, 512)
    return pl.pallas_call(
        functools.partial(_pre_body, n, tq, has_fc0),
        out_shape=[jax.ShapeDtypeStruct((n, _K), jnp.int32),
                   jax.ShapeDtypeStruct((n, dout), _F32),
                   jax.ShapeDtypeStruct((n, 128), _F32)],
    )(pos_p, posT, x, *warrs)


def _gather_body_tc(nb, tq, idx_r, tab_r, out_r):
    tab = tab_r[...]
    iota = lax.broadcasted_iota(jnp.int32, (tq, nb), 1)
    for j in range(_K):
        idq = idx_r[:, pl.ds(j, 1)]
        oh = (iota == idq).astype(_F32)
        out_r[j, :, :] = _dot(oh, tab)


def _gather_rows(nbr, tab):
    n = nbr.shape[0]
    nb, Dp = tab.shape
    tq = min(n, 512)
    return pl.pallas_call(
        functools.partial(_gather_body_tc, nb, tq),
        grid=(n // tq,),
        in_specs=[pl.BlockSpec((tq, _K), lambda t: (t, 0)),
                  pl.BlockSpec((nb, Dp), lambda t: (0, 0))],
        out_specs=pl.BlockSpec((_K, tq, Dp), lambda t: (0, t, 0)),
        out_shape=jax.ShapeDtypeStruct((_K, n, Dp), _F32),
    )(nbr, tab)


def _gather_rows_sc(nbr, tab):
    n = nbr.shape[0]
    nb, Dp = tab.shape
    B = _K * n
    info = plsc.get_sparse_core_info()
    NC, NS = info.num_cores, info.num_subcores
    NW = NC * NS
    bpw = B // NW
    ch = bpw if bpw <= 128 else 128
    nch = bpw // ch
    idx3 = nbr.T.reshape(NW, nch, ch)
    mesh = plsc.VectorSubcoreMesh(core_axis_name="c", subcore_axis_name="s")

    @functools.partial(
        pl.kernel, mesh=mesh,
        out_type=jax.ShapeDtypeStruct((B, Dp), _F32),
        scratch_types=[
            pltpu.VMEM((nch, ch), jnp.int32),
            pltpu.VMEM((2, ch, Dp), _F32),
            pltpu.SemaphoreType.DMA,
        ],
    )
    def k(tab_hbm, idx_hbm, out_hbm, idx_v, buf_v, sem):
        wid = lax.axis_index("s") * NC + lax.axis_index("c")
        base = wid * bpw
        pltpu.sync_copy(idx_hbm.at[wid], idx_v)
        descs = [pltpu.async_copy(tab_hbm.at[idx_v.at[0]], buf_v.at[0], sem)]
        for c in range(nch):
            if c + 1 < nch:
                descs.append(pltpu.async_copy(
                    tab_hbm.at[idx_v.at[c + 1]], buf_v.at[(c + 1) % 2], sem))
            descs[c].wait()
            pltpu.sync_copy(buf_v.at[c % 2],
                            out_hbm.at[pl.ds(base + c * ch, ch)])

    return k(tab, idx3).reshape(_K, n, Dp)


def _lfa_body(ce, tq, mode, g_r, pos_r, *rest):
    if mode == 'post':
        (wpi_r, wpj_r, wds_r, be_r, axx_r, axe_r, aex_r, aee_r,
         px_r, pe_r, bp_r, sc_r, wm2_r, bm2_r, out_r) = rest
    else:
        (wpi_r, wpj_r, wds_r, be_r, axx_r, axe_r, aex_r, aee_r,
         px_r, pe_r, bp_r, out_r) = rest
    pos_i = pos_r[:, 0:3]
    wpj = wpj_r[...]
    wds = wds_r[...]
    axx, axe, aex, aee = axx_r[...], axe_r[...], aex_r[...], aee_r[...]
    pit = _dot(pos_i, wpi_r[...]) + be_r[...]
    encs, xjs, axs, aes = [], [], [], []
    mx = me = None
    for j in range(_K):
        xj = g_r[j, :, 0:ce]
        pj = g_r[j, :, ce:ce + 3]
        df = pj - pos_i
        dist = jnp.sqrt(jnp.sum(df * df, axis=1, keepdims=True))
        enc = _leaky(pit + _dot(pj, wpj) + dist * wds)
        ax = _dot(xj, axx) + _dot(enc, aex)
        ae = _dot(xj, axe) + _dot(enc, aee)
        encs.append(enc)
        xjs.append(xj)
        axs.append(ax)
        aes.append(ae)
        mx = ax if mx is None else jnp.maximum(mx, ax)
        me = ae if me is None else jnp.maximum(me, ae)
    sx = se = aggx = agge = None
    for j in range(_K):
        ex = jnp.exp(axs[j] - mx)
        ee = jnp.exp(aes[j] - me)
        px_ = ex * xjs[j]
        pe_ = ee * encs[j]
        if j == 0:
            sx, se, aggx, agge = ex, ee, px_, pe_
        else:
            sx += ex
            se += ee
            aggx += px_
            agge += pe_
    res = _leaky(_dot(aggx / sx, px_r[...]) + _dot(agge / se, pe_r[...])
                 + bp_r[...])
    if mode == 'post':
        out_r[...] = _leaky(_dot(res, wm2_r[...]) + bm2_r[...] + sc_r[...])
    else:
        out_r[:, 0:2 * ce] = res
        out_r[:, 2 * ce:2 * ce + 3] = pos_i
        out_r[:, 2 * ce + 3:] = jnp.zeros((tq, 128 - 2 * ce - 3), _F32)


def _lfa(g, pos_p, lp, ce, n, sc=None, mlp2=None):
    We, be = _fold(lp['enc'][0])
    Wa, _ = _fold(lp['att'][0])
    Wp, bp = _fold(lp['post'][0])
    wpi, wpj, wdf, wds = We[0:3], We[3:6], We[6:9], We[9:10]
    wpi = wpi - wdf
    wpj = wpj + wdf
    axx, axe = Wa[:ce, :ce], Wa[:ce, ce:]
    aex, aee = Wa[ce:, :ce], Wa[ce:, ce:]
    px, pe = Wp[:ce], Wp[ce:]
    Dp = g.shape[2]
    tq = min(n, 512)
    _w = lambda a: pl.BlockSpec(a.shape, lambda t: (0,) * a.ndim)
    args = [g, pos_p, wpi, wpj, wds, be, axx, axe, aex, aee, px, pe, bp]
    specs = [pl.BlockSpec((_K, tq, Dp), lambda t: (0, t, 0)),
             pl.BlockSpec((tq, 8), lambda t: (t, 0))] + [_w(a) for a in args[2:]]
    if mlp2 is not None:
        mode = 'post'
        wm2, bm2 = mlp2
        dout = wm2.shape[1]
        args += [sc, wm2, bm2]
        specs += [pl.BlockSpec((tq, dout), lambda t: (t, 0)), _w(wm2), _w(bm2)]
        out_shape = jax.ShapeDtypeStruct((n, dout), _F32)
        out_spec = pl.BlockSpec((tq, dout), lambda t: (t, 0))
    else:
        mode = 'tab'
        out_shape = jax.ShapeDtypeStruct((n, 128), _F32)
        out_spec = pl.BlockSpec((tq, 128), lambda t: (t, 0))
    return pl.pallas_call(
        functools.partial(_lfa_body, ce, tq, mode),
        grid=(n // tq,),
        in_specs=specs,
        out_specs=out_spec,
        out_shape=out_shape,
    )(*args)


def _itp_oh(q3, bT):
    nq, nb = q3.shape[0], bT.shape[1]
    d = None
    for c in range(3):
        df = q3[:, c:c + 1] - bT[c:c + 1, :]
        d = df * df if d is None else d + df * df
    iota = lax.broadcasted_iota(jnp.int32, (nq, nb), 1)
    dmin = jnp.min(d, axis=1, keepdims=True)
    idxc = jnp.min(jnp.where(d == dmin, iota, nb), axis=1, keepdims=True)
    return (iota == idxc).astype(_F32)


def _head_body(pos_r, posT_r, f3_r, x1_r, w21_r, w22_r, b2_r,
               w11_r, w12_r, b1_r, we1_r, be1_r, we2_r, be2_r,
               we3_r, be3_r, out_r):
    pos3 = pos_r[:, 0:3]
    posT = posT_r[...]
    x1 = x1_r[...]

    f2 = _leaky(_dot(_dot(_itp_oh(pos3[:1024], posT[:, :256]), f3_r[...]),
                     w21_r[...])
                + _dot(x1[:1024], w22_r[...]) + b2_r[...])

    w11, w12, b1 = w11_r[...], w12_r[...], b1_r[...]
    we1, be1 = we1_r[...], be1_r[...]
    we2, be2 = we2_r[...], be2_r[...]
    we3, be3 = we3_r[...], be3_r[...]
    tq = 1024
    for t in range(4):
        base = t * tq
        oh = _itp_oh(pos3[base:base + tq], posT)
        f1 = _leaky(_dot(_dot(oh, f2), w11)
                    + _dot(x1[base:base + tq], w12) + b1)
        h = _leaky(_dot(f1, we1) + be1)
        h = _leaky(_dot(h, we2) + be2)
        lg = _dot(h, we3) + be3
        sh = lg - jnp.max(lg, axis=1, keepdims=True)
        out_r[pl.ds(base, tq), :] = sh - jnp.log(
            jnp.sum(jnp.exp(sh), axis=1, keepdims=True))


def _head(pos_p, posT, f3, x1, P):
    W2, b2 = _fold(P['fp2'][0])
    W1, b1 = _fold(P['fp1'][0])
    We1, be1 = _fold(P['end_mlp'][0])
    We2, be2 = _fold(P['end_mlp'][1])
    We3, be3 = _fold(P['end_lin'])
    return pl.pallas_call(
        _head_body,
        out_shape=jax.ShapeDtypeStruct((4096, 13), _F32),
    )(pos_p, posT[:, :1024], f3, x1, W2[:128], W2[128:], b2,
      W1[:32], W1[32:], b1, We1, be1, We2, be2, We3, be3)


def _knn_oh_val(pos3, posT, n):
    d = None
    for c in range(3):
        df = pos3[:, c:c + 1] - posT[c:c + 1, :]
        d = df * df if d is None else d + df * df
    iota = lax.broadcasted_iota(jnp.int32, (n, n), 1)
    ohs = []
    for j in range(_K):
        dmin = jnp.min(d, axis=1, keepdims=True)
        eqm = d == dmin
        idxc = jnp.min(jnp.where(eqm, iota, n), axis=1, keepdims=True)
        ohs.append((iota == idxc).astype(_F32))
        d = jnp.where(eqm, jnp.float32(jnp.inf), d)
    return ohs


def _lfa_val(ohs, h, pos3, w):
    wpi, wpj, wds, be, axx, axe, aex, aee, px, pe, bp = w
    pit = _dot(pos3, wpi) + be
    xjs, encs, axs, aes = [], [], [], []
    mx = me = None
    for oh in ohs:
        xj = _dot(oh, h)
        pj = _dot(oh, pos3)
        df = pj - pos3
        dist = jnp.sqrt(jnp.sum(df * df, axis=1, keepdims=True))
        enc = _leaky(pit + _dot(pj, wpj) + dist * wds)
        ax = _dot(xj, axx) + _dot(enc, aex)
        ae = _dot(xj, axe) + _dot(enc, aee)
        xjs.append(xj)
        encs.append(enc)
        axs.append(ax)
        aes.append(ae)
        mx = ax if mx is None else jnp.maximum(mx, ax)
        me = ae if me is None else jnp.maximum(me, ae)
    sx = se = aggx = agge = None
    for j in range(_K):
        ex = jnp.exp(axs[j] - mx)
        ee = jnp.exp(aes[j] - me)
        px_ = ex * xjs[j]
        pe_ = ee * encs[j]
        if j == 0:
            sx, se, aggx, agge = ex, ee, px_, pe_
        else:
            sx += ex
            se += ee
            aggx += px_
            agge += pe_
    return _leaky(_dot(aggx / sx, px) + _dot(agge / se, pe) + bp)


def _itp_val(q3, bT, feats, skip, w1, w2, b):
    nq = q3.shape[0]
    nb = bT.shape[1]
    d = None
    for c in range(3):
        df = q3[:, c:c + 1] - bT[c:c + 1, :]
        d = df * df if d is None else d + df * df
    iota = lax.broadcasted_iota(jnp.int32, (nq, nb), 1)
    dmin = jnp.min(d, axis=1, keepdims=True)
    idxc = jnp.min(jnp.where(d == dmin, iota, nb), axis=1, keepdims=True)
    oh = (iota == idxc).astype(_F32)
    return _leaky(_dot(_dot(oh, feats), w1) + _dot(skip, w2) + b)


def _lfa_wlist(lp, ce):
    We, be = _fold(lp['enc'][0])
    Wa, _ = _fold(lp['att'][0])
    Wp, bp = _fold(lp['post'][0])
    wpi, wpj, wdf, wds = We[0:3], We[3:6], We[6:9], We[9:10]
    return [wpi - wdf, wpj + wdf, wds, be,
            Wa[:ce, :ce], Wa[:ce, ce:], Wa[ce:, :ce], Wa[ce:, ce:],
            Wp[:ce], Wp[ce:], bp]


def _tail_body(x_r, pos_r, posT_r, *rest):
    out_r = rest[-1]
    it = iter(rest[:-1])
    nxt = lambda: next(it)[...]

    pos3 = pos_r[:, 0:3]
    posT = posT_r[...]

    def blkv(x, n, dout):
        p3 = pos3[:n]
        ohs = _knn_oh_val(p3, posT[:, :n], n)
        Ws, bs, Wm, bm = nxt(), nxt(), nxt(), nxt()
        sc = _dot(x, Ws) + bs
        h = _leaky(_dot(x, Wm) + bm)
        h = _lfa_val(ohs, h, p3, [nxt() for _ in range(11)])
        h = _lfa_val(ohs, h, p3, [nxt() for _ in range(11)])
        Wm2, bm2 = nxt(), nxt()
        return _leaky(_dot(h, Wm2) + bm2 + sc)

    x2d = x_r[...]
    x3 = blkv(x2d, 256, 256)
    x4 = blkv(x3[:64], 64, 512)
    xs = _leaky(_dot(x4[:16], nxt()) + nxt())
    f4 = _itp_val(pos3[:64], posT[:, :16], xs, x3[:64], nxt(), nxt(), nxt())
    f3 = _itp_val(pos3, posT[:, :64], f4, x2d, nxt(), nxt(), nxt())
    out_r[...] = f3


def _tail(x2d, pos_p, posT, P):
    def lin(p):
        W, b = _fold(p)
        return [W, b]

    def blk(bp, dout):
        return (lin(bp['shortcut'][0]) + lin(bp['mlp1'][0])
                + _lfa_wlist(bp['lfa1'], dout // 8)
                + _lfa_wlist(bp['lfa2'], dout // 4)
                + lin(bp['mlp2'][0]))

    def fpw(mp, cf):
        W, b = _fold(mp[0])
        return [W[:cf], W[cf:], b]

    arrs = (blk(P['b3'], 256) + blk(P['b4'], 512) + lin(P['summit'][0])
            + fpw(P['fp4'], 512) + fpw(P['fp3'], 256))
    return pl.pallas_call(
        _tail_body,
        out_shape=jax.ShapeDtypeStruct((256, 128), _F32),
    )(x2d, pos_p[:256], posT[:, :256], *arrs)


def kernel(x, pos, batch, ptr, params):
    P = params
    pos_p = jnp.pad(pos, ((0, 0), (0, 5)))
    posT = jnp.pad(pos.T, ((0, 5), (0, 0)))

    def block(bp, xin, n, dout, fc0=None):
        posn = pos_p[:n]
        posTn = posT[:, :n]
        warrs = []
        if fc0 is not None:
            Wf, bf = _fold(fc0)
            warrs += [Wf, bf]
        Ws, bs = _fold(bp['shortcut'][0])
        Wm, bm = _fold(bp['mlp1'][0])
        warrs += [Ws, bs, Wm, bm]
        nbr, sc, tab1 = _pre(posn, posTn, xin, warrs, n, dout, fc0 is not None)
        g1 = _gather_rows(nbr, tab1)
        tab2 = _lfa(g1, posn, bp['lfa1'], dout // 8, n)
        g2 = _gather_rows(nbr, tab2)
        return _lfa(g2, posn, bp['lfa2'], dout // 4, n,
                    sc=sc, mlp2=_fold(bp['mlp2'][0]))

    x1 = block(P['b1'], x, 4096, 32, fc0=P['fc0'])
    x2 = block(P['b2'], x1[:1024], 1024, 128)
    f3 = _tail(x2[:256], pos_p, posT, P)
    return _head(pos_p, posT, f3, x1, P)


# speedup vs baseline: 1.2872x; 1.2872x over previous
import functools

import jax
import jax.numpy as jnp
import numpy as np
from jax import lax
from jax.experimental import pallas as pl
from jax.experimental.pallas import tpu as pltpu
from jax.experimental.pallas import tpu_sc as plsc

_K = 16
_NS = float(1.0 / np.sqrt(1.0 + 1e-5))
_F32 = jnp.float32


def _leaky(v):
    return jnp.where(v >= 0, v, v * 0.2)


def _fold(p):
    W = p['W']
    b = p.get('b')
    if 'gamma' in p:
        g = p['gamma'] * _NS
        W = W * g[None, :]
        b = (b * g + p['beta']) if b is not None else p['beta']
    return W, (None if b is None else b.reshape(1, -1))


def _dot(a, b):
    return jnp.dot(a, b, preferred_element_type=_F32)


def _pre_body(n, tq, has_fc0, pos_r, posT_r, x_r, *rest):
    nbr_r, sc_r, tab_r = rest[-3], rest[-2], rest[-1]
    it = iter(rest[:-3])
    nxt = lambda: next(it)[...]

    x = x_r[...]
    if has_fc0:
        x = _dot(x, nxt()) + nxt()
    sc_r[...] = _dot(x, nxt()) + nxt()
    h = _leaky(_dot(x, nxt()) + nxt())
    ch = h.shape[1]
    tab_r[:, 0:ch] = h
    tab_r[:, ch:ch + 3] = pos_r[:, 0:3]
    tab_r[:, ch + 3:] = jnp.zeros((n, 128 - ch - 3), _F32)

    iota = lax.broadcasted_iota(jnp.int32, (tq, n), 1)

    def tile(t, carry):
        base = t * tq
        q = pos_r[pl.ds(base, tq), :]
        d = None
        for c in range(3):
            df = q[:, c:c + 1] - posT_r[c:c + 1, :]
            d = df * df if d is None else d + df * df
        for j in range(_K):
            dmin = jnp.min(d, axis=1, keepdims=True)
            eqm = d == dmin
            idxc = jnp.min(jnp.where(eqm, iota, n), axis=1, keepdims=True)
            nbr_r[pl.ds(base, tq), pl.ds(j, 1)] = idxc
            d = jnp.where(eqm, jnp.float32(jnp.inf), d)
        return carry

    lax.fori_loop(0, n // tq, tile, 0)


def _pre(pos_p, posT, x, warrs, n, dout, has_fc0):
    tq = min(n, 512)
    return pl.pallas_call(
        functools.partial(_pre_body, n, tq, has_fc0),
        out_shape=[jax.ShapeDtypeStruct((n, _K), jnp.int32),
                   jax.ShapeDtypeStruct((n, dout), _F32),
                   jax.ShapeDtypeStruct((n, 128), _F32)],
    )(pos_p, posT, x, *warrs)


def _gather_rows(nbr, tab):
    n = nbr.shape[0]
    nb, Dp = tab.shape
    B = _K * n
    info = plsc.get_sparse_core_info()
    NC, NS = info.num_cores, info.num_subcores
    NW = NC * NS
    bpw = B // NW
    ch = bpw if bpw <= 128 else 128
    nch = bpw // ch
    nbuf = min(nch, 6)
    idx3 = nbr.T.reshape(NW, nch, ch)
    mesh = plsc.VectorSubcoreMesh(core_axis_name="c", subcore_axis_name="s")

    @functools.partial(
        pl.kernel, mesh=mesh,
        out_type=jax.ShapeDtypeStruct((B, Dp), _F32),
        scratch_types=[
            pltpu.VMEM((nch, ch), jnp.int32),
            pltpu.VMEM((nbuf, ch, Dp), _F32),
            pltpu.SemaphoreType.DMA,
            pltpu.SemaphoreType.DMA,
        ],
    )
    def k(tab_hbm, idx_hbm, out_hbm, idx_v, buf_v, gsem, wsem):
        wid = lax.axis_index("s") * NC + lax.axis_index("c")
        base = wid * bpw
        pltpu.sync_copy(idx_hbm.at[wid], idx_v)
        g = [None] * nch
        w = [None] * nch
        for c in range(nbuf):
            g[c] = pltpu.async_copy(tab_hbm.at[idx_v.at[c]],
                                    buf_v.at[c], gsem)
        for c in range(nch):
            g[c].wait()
            w[c] = pltpu.async_copy(buf_v.at[c % nbuf],
                                    out_hbm.at[pl.ds(base + c * ch, ch)], wsem)
            nx = c + nbuf
            if nx < nch:
                w[c].wait()
                g[nx] = pltpu.async_copy(tab_hbm.at[idx_v.at[nx]],
                                         buf_v.at[nx % nbuf], gsem)
        for c in range(max(0, nch - nbuf), nch):
            w[c].wait()

    return k(tab, idx3).reshape(_K, n, Dp)


def _lfa_body(ce, tq, mode, g_r, pos_r, *rest):
    if mode == 'post':
        (wpi_r, wpj_r, wds_r, be_r, axx_r, axe_r, aex_r, aee_r,
         px_r, pe_r, bp_r, sc_r, wm2_r, bm2_r, out_r) = rest
    else:
        (wpi_r, wpj_r, wds_r, be_r, axx_r, axe_r, aex_r, aee_r,
         px_r, pe_r, bp_r, out_r) = rest
    pos_i = pos_r[:, 0:3]
    wpj = wpj_r[...]
    wds = wds_r[...]
    axx, axe, aex, aee = axx_r[...], axe_r[...], aex_r[...], aee_r[...]
    pit = _dot(pos_i, wpi_r[...]) + be_r[...]
    encs, xjs, axs, aes = [], [], [], []
    mx = me = None
    for j in range(_K):
        xj = g_r[j, :, 0:ce]
        pj = g_r[j, :, ce:ce + 3]
        df = pj - pos_i
        dist = jnp.sqrt(jnp.sum(df * df, axis=1, keepdims=True))
        enc = _leaky(pit + _dot(pj, wpj) + dist * wds)
        ax = _dot(xj, axx) + _dot(enc, aex)
        ae = _dot(xj, axe) + _dot(enc, aee)
        encs.append(enc)
        xjs.append(xj)
        axs.append(ax)
        aes.append(ae)
        mx = ax if mx is None else jnp.maximum(mx, ax)
        me = ae if me is None else jnp.maximum(me, ae)
    sx = se = aggx = agge = None
    for j in range(_K):
        ex = jnp.exp(axs[j] - mx)
        ee = jnp.exp(aes[j] - me)
        px_ = ex * xjs[j]
        pe_ = ee * encs[j]
        if j == 0:
            sx, se, aggx, agge = ex, ee, px_, pe_
        else:
            sx += ex
            se += ee
            aggx += px_
            agge += pe_
    res = _leaky(_dot(aggx / sx, px_r[...]) + _dot(agge / se, pe_r[...])
                 + bp_r[...])
    if mode == 'post':
        out_r[...] = _leaky(_dot(res, wm2_r[...]) + bm2_r[...] + sc_r[...])
    else:
        out_r[:, 0:2 * ce] = res
        out_r[:, 2 * ce:2 * ce + 3] = pos_i
        out_r[:, 2 * ce + 3:] = jnp.zeros((tq, 128 - 2 * ce - 3), _F32)


def _lfa(g, pos_p, lp, ce, n, sc=None, mlp2=None):
    We, be = _fold(lp['enc'][0])
    Wa, _ = _fold(lp['att'][0])
    Wp, bp = _fold(lp['post'][0])
    wpi, wpj, wdf, wds = We[0:3], We[3:6], We[6:9], We[9:10]
    wpi = wpi - wdf
    wpj = wpj + wdf
    axx, axe = Wa[:ce, :ce], Wa[:ce, ce:]
    aex, aee = Wa[ce:, :ce], Wa[ce:, ce:]
    px, pe = Wp[:ce], Wp[ce:]
    Dp = g.shape[2]
    tq = min(n, 512)
    _w = lambda a: pl.BlockSpec(a.shape, lambda t: (0,) * a.ndim)
    args = [g, pos_p, wpi, wpj, wds, be, axx, axe, aex, aee, px, pe, bp]
    specs = [pl.BlockSpec((_K, tq, Dp), lambda t: (0, t, 0)),
             pl.BlockSpec((tq, 8), lambda t: (t, 0))] + [_w(a) for a in args[2:]]
    if mlp2 is not None:
        mode = 'post'
        wm2, bm2 = mlp2
        dout = wm2.shape[1]
        args += [sc, wm2, bm2]
        specs += [pl.BlockSpec((tq, dout), lambda t: (t, 0)), _w(wm2), _w(bm2)]
        out_shape = jax.ShapeDtypeStruct((n, dout), _F32)
        out_spec = pl.BlockSpec((tq, dout), lambda t: (t, 0))
    else:
        mode = 'tab'
        out_shape = jax.ShapeDtypeStruct((n, 128), _F32)
        out_spec = pl.BlockSpec((tq, 128), lambda t: (t, 0))
    return pl.pallas_call(
        functools.partial(_lfa_body, ce, tq, mode),
        grid=(n // tq,),
        in_specs=specs,
        out_specs=out_spec,
        out_shape=out_shape,
    )(*args)


def _itp_oh(q3, bT):
    nq, nb = q3.shape[0], bT.shape[1]
    d = None
    for c in range(3):
        df = q3[:, c:c + 1] - bT[c:c + 1, :]
        d = df * df if d is None else d + df * df
    iota = lax.broadcasted_iota(jnp.int32, (nq, nb), 1)
    dmin = jnp.min(d, axis=1, keepdims=True)
    idxc = jnp.min(jnp.where(d == dmin, iota, nb), axis=1, keepdims=True)
    return (iota == idxc).astype(_F32)


def _head_body(pos_r, posT_r, f3_r, x1_r, w21_r, w22_r, b2_r,
               w11_r, w12_r, b1_r, we1_r, be1_r, we2_r, be2_r,
               we3_r, be3_r, out_r):
    pos3 = pos_r[:, 0:3]
    posT = posT_r[...]
    x1 = x1_r[...]

    f2 = _leaky(_dot(_dot(_itp_oh(pos3[:1024], posT[:, :256]), f3_r[...]),
                     w21_r[...])
                + _dot(x1[:1024], w22_r[...]) + b2_r[...])

    w11, w12, b1 = w11_r[...], w12_r[...], b1_r[...]
    we1, be1 = we1_r[...], be1_r[...]
    we2, be2 = we2_r[...], be2_r[...]
    we3, be3 = we3_r[...], be3_r[...]
    tq = 1024
    for t in range(4):
        base = t * tq
        oh = _itp_oh(pos3[base:base + tq], posT)
        f1 = _leaky(_dot(_dot(oh, f2), w11)
                    + _dot(x1[base:base + tq], w12) + b1)
        h = _leaky(_dot(f1, we1) + be1)
        h = _leaky(_dot(h, we2) + be2)
        lg = _dot(h, we3) + be3
        sh = lg - jnp.max(lg, axis=1, keepdims=True)
        out_r[pl.ds(base, tq), :] = sh - jnp.log(
            jnp.sum(jnp.exp(sh), axis=1, keepdims=True))


def _head(pos_p, posT, f3, x1, P):
    W2, b2 = _fold(P['fp2'][0])
    W1, b1 = _fold(P['fp1'][0])
    We1, be1 = _fold(P['end_mlp'][0])
    We2, be2 = _fold(P['end_mlp'][1])
    We3, be3 = _fold(P['end_lin'])
    return pl.pallas_call(
        _head_body,
        out_shape=jax.ShapeDtypeStruct((4096, 13), _F32),
    )(pos_p, posT[:, :1024], f3, x1, W2[:128], W2[128:], b2,
      W1[:32], W1[32:], b1, We1, be1, We2, be2, We3, be3)


def _knn_oh_val(pos3, posT, n):
    d = None
    for c in range(3):
        df = pos3[:, c:c + 1] - posT[c:c + 1, :]
        d = df * df if d is None else d + df * df
    iota = lax.broadcasted_iota(jnp.int32, (n, n), 1)
    ohs = []
    for j in range(_K):
        dmin = jnp.min(d, axis=1, keepdims=True)
        eqm = d == dmin
        idxc = jnp.min(jnp.where(eqm, iota, n), axis=1, keepdims=True)
        ohs.append((iota == idxc).astype(_F32))
        d = jnp.where(eqm, jnp.float32(jnp.inf), d)
    return ohs


def _lfa_val(ohs, h, pos3, w):
    wpi, wpj, wds, be, axx, axe, aex, aee, px, pe, bp = w
    pit = _dot(pos3, wpi) + be
    xjs, encs, axs, aes = [], [], [], []
    mx = me = None
    for oh in ohs:
        xj = _dot(oh, h)
        pj = _dot(oh, pos3)
        df = pj - pos3
        dist = jnp.sqrt(jnp.sum(df * df, axis=1, keepdims=True))
        enc = _leaky(pit + _dot(pj, wpj) + dist * wds)
        ax = _dot(xj, axx) + _dot(enc, aex)
        ae = _dot(xj, axe) + _dot(enc, aee)
        xjs.append(xj)
        encs.append(enc)
        axs.append(ax)
        aes.append(ae)
        mx = ax if mx is None else jnp.maximum(mx, ax)
        me = ae if me is None else jnp.maximum(me, ae)
    sx = se = aggx = agge = None
    for j in range(_K):
        ex = jnp.exp(axs[j] - mx)
        ee = jnp.exp(aes[j] - me)
        px_ = ex * xjs[j]
        pe_ = ee * encs[j]
        if j == 0:
            sx, se, aggx, agge = ex, ee, px_, pe_
        else:
            sx += ex
            se += ee
            aggx += px_
            agge += pe_
    return _leaky(_dot(aggx / sx, px) + _dot(agge / se, pe) + bp)


def _itp_val(q3, bT, feats, skip, w1, w2, b):
    nq = q3.shape[0]
    nb = bT.shape[1]
    d = None
    for c in range(3):
        df = q3[:, c:c + 1] - bT[c:c + 1, :]
        d = df * df if d is None else d + df * df
    iota = lax.broadcasted_iota(jnp.int32, (nq, nb), 1)
    dmin = jnp.min(d, axis=1, keepdims=True)
    idxc = jnp.min(jnp.where(d == dmin, iota, nb), axis=1, keepdims=True)
    oh = (iota == idxc).astype(_F32)
    return _leaky(_dot(_dot(oh, feats), w1) + _dot(skip, w2) + b)


def _lfa_wlist(lp, ce):
    We, be = _fold(lp['enc'][0])
    Wa, _ = _fold(lp['att'][0])
    Wp, bp = _fold(lp['post'][0])
    wpi, wpj, wdf, wds = We[0:3], We[3:6], We[6:9], We[9:10]
    return [wpi - wdf, wpj + wdf, wds, be,
            Wa[:ce, :ce], Wa[:ce, ce:], Wa[ce:, :ce], Wa[ce:, ce:],
            Wp[:ce], Wp[ce:], bp]


def _tail_body(x_r, pos_r, posT_r, *rest):
    out_r = rest[-1]
    it = iter(rest[:-1])
    nxt = lambda: next(it)[...]

    pos3 = pos_r[:, 0:3]
    posT = posT_r[...]

    def blkv(x, n, dout):
        p3 = pos3[:n]
        ohs = _knn_oh_val(p3, posT[:, :n], n)
        Ws, bs, Wm, bm = nxt(), nxt(), nxt(), nxt()
        sc = _dot(x, Ws) + bs
        h = _leaky(_dot(x, Wm) + bm)
        h = _lfa_val(ohs, h, p3, [nxt() for _ in range(11)])
        h = _lfa_val(ohs, h, p3, [nxt() for _ in range(11)])
        Wm2, bm2 = nxt(), nxt()
        return _leaky(_dot(h, Wm2) + bm2 + sc)

    x2d = x_r[...]
    x3 = blkv(x2d, 256, 256)
    x4 = blkv(x3[:64], 64, 512)
    xs = _leaky(_dot(x4[:16], nxt()) + nxt())
    f4 = _itp_val(pos3[:64], posT[:, :16], xs, x3[:64], nxt(), nxt(), nxt())
    f3 = _itp_val(pos3, posT[:, :64], f4, x2d, nxt(), nxt(), nxt())
    out_r[...] = f3


def _tail(x2d, pos_p, posT, P):
    def lin(p):
        W, b = _fold(p)
        return [W, b]

    def blk(bp, dout):
        return (lin(bp['shortcut'][0]) + lin(bp['mlp1'][0])
                + _lfa_wlist(bp['lfa1'], dout // 8)
                + _lfa_wlist(bp['lfa2'], dout // 4)
                + lin(bp['mlp2'][0]))

    def fpw(mp, cf):
        W, b = _fold(mp[0])
        return [W[:cf], W[cf:], b]

    arrs = (blk(P['b3'], 256) + blk(P['b4'], 512) + lin(P['summit'][0])
            + fpw(P['fp4'], 512) + fpw(P['fp3'], 256))
    return pl.pallas_call(
        _tail_body,
        out_shape=jax.ShapeDtypeStruct((256, 128), _F32),
    )(x2d, pos_p[:256], posT[:, :256], *arrs)


def kernel(x, pos, batch, ptr, params):
    P = params
    pos_p = jnp.pad(pos, ((0, 0), (0, 5)))
    posT = jnp.pad(pos.T, ((0, 5), (0, 0)))

    def block(bp, xin, n, dout, fc0=None):
        posn = pos_p[:n]
        posTn = posT[:, :n]
        warrs = []
        if fc0 is not None:
            Wf, bf = _fold(fc0)
            warrs += [Wf, bf]
        Ws, bs = _fold(bp['shortcut'][0])
        Wm, bm = _fold(bp['mlp1'][0])
        warrs += [Ws, bs, Wm, bm]
        nbr, sc, tab1 = _pre(posn, posTn, xin, warrs, n, dout, fc0 is not None)
        g1 = _gather_rows(nbr, tab1)
        tab2 = _lfa(g1, posn, bp['lfa1'], dout // 8, n)
        g2 = _gather_rows(nbr, tab2)
        return _lfa(g2, posn, bp['lfa2'], dout // 4, n,
                    sc=sc, mlp2=_fold(bp['mlp2'][0]))

    x1 = block(P['b1'], x, 4096, 32, fc0=P['fc0'])
    x2 = block(P['b2'], x1[:1024], 1024, 128)
    f3 = _tail(x2[:256], pos_p, posT, P)
    return _head(pos_p, posT, f3, x1, P)
